# baseline (speedup 1.0000x reference)
import jax
import jax.numpy as jnp
from jax import lax
from jax.experimental import pallas as pl
from jax.experimental.pallas import tpu as pltpu
from jax.experimental.pallas import tpu_sc as plsc

N = 10000
E = 320000
D = 128

NC = 2
NS = 16
NW = NC * NS
CHUNK = 96
CPT = -(-E // (NW * CHUNK))
E_PAD = NW * CPT * CHUNK
N_PAD = 10240
ZPT = N_PAD // NS


def _sc_body(x_hbm, src_hbm, dst_hbm, w_hbm, zeros_hbm, out_hbm,
             srcs, dsts, w_v, rows_ab, acc,
             gsem, wsem, ssem, isem):
    c = lax.axis_index("c")
    s = lax.axis_index("s")
    g = c * NS + s

    r0 = s * ZPT
    pltpu.sync_copy(src_hbm.at[g, 0], srcs[0])
    pltpu.async_copy(w_hbm.at[g, 0], w_v, wsem)
    pltpu.async_copy(x_hbm.at[srcs[0]], rows_ab[0], gsem)
    pltpu.sync_copy(dst_hbm.at[g, 0], dsts[0])
    pltpu.sync_copy(src_hbm.at[g, 1], srcs[1])
    pltpu.sync_copy(dst_hbm.at[g, 1], dsts[1])
    pltpu.sync_copy(zeros_hbm.at[pl.ds(r0, ZPT)], acc.at[pl.ds(r0, ZPT)])
    plsc.subcore_barrier()

    def chunk_body(i, carry):
        par = lax.rem(i, 2)
        npar = 1 - par
        slot = lax.rem(i, 3)
        slot_n = lax.rem(i + 1, 3)
        slot_nn = lax.rem(i + 2, 3)

        @pl.when(i >= 1)
        def _drain_prev_scatter():
            pltpu.make_async_copy(rows_ab[0], acc.at[dsts[0]], ssem).wait()

        @pl.when((i >= 1) & (i + 1 < CPT))
        def _wait_next_idx():
            pltpu.make_async_copy(src_hbm.at[g, i + 1], srcs[0], isem).wait()
            pltpu.make_async_copy(dst_hbm.at[g, i + 1], dsts[0], isem).wait()

        pltpu.make_async_copy(w_hbm.at[g, i], w_v, wsem).wait()
        pltpu.make_async_copy(x_hbm.at[srcs[0]], rows_ab[0], gsem).wait()

        @pl.when(i + 1 < CPT)
        def _prefetch_next():
            for k in range(3):
                @pl.when(slot_n == k)
                def _(k=k):
                    for p in range(2):
                        @pl.when(npar == p)
                        def _(k=k, p=p):
                            pltpu.async_copy(x_hbm.at[srcs[k]], rows_ab[p],
                                             gsem)

        @pl.when(i + 2 < CPT)
        def _fetch_idx_two_ahead():
            for k in range(3):
                @pl.when(slot_nn == k)
                def _(k=k):
                    pltpu.async_copy(src_hbm.at[g, i + 2], srcs[k], isem)
                    pltpu.async_copy(dst_hbm.at[g, i + 2], dsts[k], isem)

        for p in range(2):
            @pl.when(par == p)
            def _mul(p=p):
                rows_v = rows_ab[p]

                def mul_body(grp, carry2):
                    for l in range(16):
                        e = grp * 16 + l
                        wb = w_v[e]
                        for j in range(D // 16):
                            sl = rows_v[e, pl.ds(j * 16, 16)]
                            rows_v[e, pl.ds(j * 16, 16)] = sl * wb
                    return carry2

                lax.fori_loop(0, CHUNK // 16, mul_body, 0, unroll=False)

        @pl.when(i + 1 < CPT)
        def _prefetch_w():
            pltpu.async_copy(w_hbm.at[g, i + 1], w_v, wsem)

        for k in range(3):
            @pl.when(slot == k)
            def _(k=k):
                for p in range(2):
                    @pl.when(par == p)
                    def _(k=k, p=p):
                        pltpu.async_copy(rows_ab[p], acc.at[dsts[k]], ssem,
                                         add=True)
        return carry

    lax.fori_loop(0, CPT, chunk_body, 0, unroll=False)
    pltpu.make_async_copy(rows_ab[0], acc.at[dsts[0]], ssem).wait()
    plsc.subcore_barrier()

    pltpu.sync_copy(acc.at[pl.ds(r0, ZPT)], out_hbm.at[c, pl.ds(r0, ZPT)])


def _add_body(a_ref, b_ref, o_ref):
    o_ref[...] = a_ref[...] + b_ref[...]


def kernel(x, edge_index, edge_weight):
    src = edge_index[0].astype(jnp.int32)
    dst = edge_index[1].astype(jnp.int32)
    pad = E_PAD - E
    src = jnp.pad(src, (0, pad)).reshape(NW, CPT, CHUNK)
    dst = jnp.pad(dst, (0, pad)).reshape(NW, CPT, CHUNK)
    wf = jnp.pad(edge_weight.astype(jnp.float32), (0, pad))
    w = jnp.broadcast_to(wf[:, None], (E_PAD, 16)).reshape(NW, CPT, CHUNK, 16)
    zeros = jnp.zeros((N_PAD, D), jnp.float32)

    sc = pl.kernel(
        _sc_body,
        out_type=jax.ShapeDtypeStruct((NC, N_PAD, D), jnp.float32),
        mesh=plsc.VectorSubcoreMesh(core_axis_name="c", subcore_axis_name="s"),
        scratch_types=[
            [pltpu.VMEM((CHUNK,), jnp.int32) for _ in range(3)],
            [pltpu.VMEM((CHUNK,), jnp.int32) for _ in range(3)],
            pltpu.VMEM((CHUNK, 16), jnp.float32),
            [pltpu.VMEM((CHUNK, D), jnp.float32) for _ in range(2)],
            pltpu.VMEM_SHARED((N_PAD, D), jnp.float32),
            pltpu.SemaphoreType.DMA,
            pltpu.SemaphoreType.DMA,
            pltpu.SemaphoreType.DMA,
            pltpu.SemaphoreType.DMA,
        ],
    )
    partial = sc(x, src, dst, w, zeros)

    blk = 2000
    out = pl.pallas_call(
        _add_body,
        grid=(N // blk,),
        in_specs=[pl.BlockSpec((blk, D), lambda i: (i, 0))] * 2,
        out_specs=pl.BlockSpec((blk, D), lambda i: (i, 0)),
        out_shape=jax.ShapeDtypeStruct((N, D), jnp.float32),
    )(partial[0], partial[1])
    return out


# speedup vs baseline: 1.3949x; 1.3949x over previous
import jax
import jax.numpy as jnp
from jax import lax
from jax.experimental import pallas as pl
from jax.experimental.pallas import tpu as pltpu
from jax.experimental.pallas import tpu_sc as plsc

N = 10000
E = 320000
D = 128

NC = 2
NS = 16
NW = NC * NS
CHUNK = 80
CPT = -(-E // (NW * CHUNK))
E_PAD = NW * CPT * CHUNK
N_PAD = 10240
ZPT = N_PAD // NS


def _sc_body(x_hbm, src_hbm, dst_hbm, w_hbm, zeros_hbm, out_hbm,
             srcs, dsts, w2_v, rows_ab, acc,
             gsem, wsem, ssem, isem):
    c = lax.axis_index("c")
    s = lax.axis_index("s")
    g = c * NS + s

    r0 = s * ZPT
    pltpu.sync_copy(src_hbm.at[g, 0], srcs[0])
    pltpu.async_copy(w_hbm.at[g, 0], w2_v.at[0], wsem)
    pltpu.async_copy(x_hbm.at[srcs[0]], rows_ab[0], gsem)
    pltpu.sync_copy(dst_hbm.at[g, 0], dsts[0])
    pltpu.sync_copy(src_hbm.at[g, 1], srcs[1])
    pltpu.sync_copy(dst_hbm.at[g, 1], dsts[1])
    pltpu.sync_copy(zeros_hbm.at[pl.ds(r0, ZPT)], acc.at[pl.ds(r0, ZPT)])
    plsc.subcore_barrier()

    def chunk_body(i, carry):
        par = lax.rem(i, 2)
        npar = 1 - par
        slot = lax.rem(i, 3)
        slot_n = lax.rem(i + 1, 3)
        slot_nn = lax.rem(i + 2, 3)

        @pl.when(i >= 1)
        def _drain_prev_scatter():
            pltpu.make_async_copy(rows_ab[0], acc.at[dsts[0]], ssem).wait()

        @pl.when((i >= 1) & (i + 1 < CPT))
        def _wait_next_idx():
            pltpu.make_async_copy(src_hbm.at[g, i + 1], srcs[0], isem).wait()
            pltpu.make_async_copy(dst_hbm.at[g, i + 1], dsts[0], isem).wait()

        pltpu.make_async_copy(w_hbm.at[g, i], w2_v.at[0], wsem).wait()
        pltpu.make_async_copy(x_hbm.at[srcs[0]], rows_ab[0], gsem).wait()

        @pl.when(i + 1 < CPT)
        def _prefetch_next():
            for k in range(3):
                @pl.when(slot_n == k)
                def _(k=k):
                    for p in range(2):
                        @pl.when(npar == p)
                        def _(k=k, p=p):
                            pltpu.async_copy(x_hbm.at[srcs[k]], rows_ab[p],
                                             gsem)
            pltpu.async_copy(w_hbm.at[g, i + 1], w2_v.at[npar], wsem)

        @pl.when(i + 2 < CPT)
        def _fetch_idx_two_ahead():
            for k in range(3):
                @pl.when(slot_nn == k)
                def _(k=k):
                    pltpu.async_copy(src_hbm.at[g, i + 2], srcs[k], isem)
                    pltpu.async_copy(dst_hbm.at[g, i + 2], dsts[k], isem)

        for p in range(2):
            @pl.when(par == p)
            def _mul(p=p):
                rows_v = rows_ab[p]
                wbuf = p

                def mul_body(grp, carry2):
                    for l in range(16):
                        e = grp * 16 + l
                        wb = w2_v[wbuf, e]
                        for j in range(D // 16):
                            sl = rows_v[e, pl.ds(j * 16, 16)]
                            rows_v[e, pl.ds(j * 16, 16)] = sl * wb
                    return carry2

                lax.fori_loop(0, CHUNK // 16, mul_body, 0, unroll=False)

        for k in range(3):
            @pl.when(slot == k)
            def _(k=k):
                for p in range(2):
                    @pl.when(par == p)
                    def _(k=k, p=p):
                        pltpu.async_copy(rows_ab[p], acc.at[dsts[k]], ssem,
                                         add=True)
        return carry

    lax.fori_loop(0, CPT, chunk_body, 0, unroll=False)
    pltpu.make_async_copy(rows_ab[0], acc.at[dsts[0]], ssem).wait()
    plsc.subcore_barrier()

    pltpu.sync_copy(acc.at[pl.ds(r0, ZPT)], out_hbm.at[c, pl.ds(r0, ZPT)])


def _add_body(a_ref, b_ref, o_ref):
    o_ref[...] = a_ref[...] + b_ref[...]


def kernel(x, edge_index, edge_weight):
    src = edge_index[0].astype(jnp.int32)
    dst = edge_index[1].astype(jnp.int32)
    pad = E_PAD - E
    src = jnp.pad(src, (0, pad)).reshape(NW, CPT, CHUNK)
    dst = jnp.pad(dst, (0, pad)).reshape(NW, CPT, CHUNK)
    wf = jnp.pad(edge_weight.astype(jnp.float32), (0, pad))
    w = jnp.broadcast_to(wf[:, None], (E_PAD, 16)).reshape(NW, CPT, CHUNK, 16)
    zeros = jnp.zeros((N_PAD, D), jnp.float32)

    sc = pl.kernel(
        _sc_body,
        out_type=jax.ShapeDtypeStruct((NC, N_PAD, D), jnp.float32),
        mesh=plsc.VectorSubcoreMesh(core_axis_name="c", subcore_axis_name="s"),
        scratch_types=[
            [pltpu.VMEM((CHUNK,), jnp.int32) for _ in range(3)],
            [pltpu.VMEM((CHUNK,), jnp.int32) for _ in range(3)],
            pltpu.VMEM((2, CHUNK, 16), jnp.float32),
            [pltpu.VMEM((CHUNK, D), jnp.float32) for _ in range(2)],
            pltpu.VMEM_SHARED((N_PAD, D), jnp.float32),
            pltpu.SemaphoreType.DMA,
            pltpu.SemaphoreType.DMA,
            pltpu.SemaphoreType.DMA,
            pltpu.SemaphoreType.DMA,
        ],
    )
    partial = sc(x, src, dst, w, zeros)

    blk = 2000
    out = pl.pallas_call(
        _add_body,
        grid=(N // blk,),
        in_specs=[pl.BlockSpec((blk, D), lambda i: (i, 0))] * 2,
        out_specs=pl.BlockSpec((blk, D), lambda i: (i, 0)),
        out_shape=jax.ShapeDtypeStruct((N, D), jnp.float32),
    )(partial[0], partial[1])
    return out
